# Initial kernel scaffold; baseline (speedup 1.0000x reference)
import jax
import jax.numpy as jnp
from jax.experimental import pallas as pl


def kernel(x, edge_index, batch, W_in, b_in, W1, b1, W2, b2, gamma, beta, Wm1, bm1, Wm2, bm2):
    raise NotImplementedError("write your pallas kernel here")


# speedup vs baseline: 6.4628x; 6.4628x over previous
import functools

import jax
import jax.numpy as jnp
from jax import lax
from jax.experimental import pallas as pl
from jax.experimental.pallas import tpu as pltpu
from jax.experimental.pallas import tpu_sc as plsc

N_NODES_C = 10000
N_EDGES_C = 320000
D_C = 128
N_GRAPHS_C = 64
EPS_BN = 1e-5

NC = 2
NS = 16
NW = NC * NS
EDGES_PER_W = N_EDGES_C // NW
CHUNK = 80
CHUNKS_PER_W = EDGES_PER_W // CHUNK
STRIPE_OFF = 624
STRIPE_LEN = 640


def _agg_body(h_hbm, src_hbm, dst_hbm, zeros_hbm, out_hbm,
              src_v, dst_v, rows_v, acc_sh, sem):
    c = lax.axis_index("c")
    s = lax.axis_index("s")
    wid = s * NC + c

    pltpu.sync_copy(zeros_hbm.at[pl.ds(s * STRIPE_OFF, STRIPE_LEN)],
                    acc_sh.at[pl.ds(s * STRIPE_OFF, STRIPE_LEN)])

    pltpu.sync_copy(src_hbm.at[wid], src_v)
    pltpu.sync_copy(dst_hbm.at[wid], dst_v)

    plsc.subcore_barrier()

    def body(i, carry):
        pltpu.async_copy(h_hbm.at[src_v.at[i]], rows_v, sem).wait()
        pltpu.sync_copy(rows_v, acc_sh.at[dst_v.at[i]], add=True)
        return carry

    lax.fori_loop(0, CHUNKS_PER_W, body, 0, unroll=False)

    plsc.subcore_barrier()

    pltpu.sync_copy(acc_sh.at[pl.ds(s * STRIPE_OFF, STRIPE_LEN)],
                    out_hbm.at[c, pl.ds(s * STRIPE_OFF, STRIPE_LEN)])


@functools.lru_cache(maxsize=None)
def _make_agg_sc():
    return pl.kernel(
        _agg_body,
        out_type=jax.ShapeDtypeStruct((NC, N_NODES_C, D_C), jnp.float32),
        mesh=plsc.VectorSubcoreMesh(core_axis_name="c", subcore_axis_name="s"),
        scratch_types=[
            pltpu.VMEM((CHUNKS_PER_W, CHUNK), jnp.int32),
            pltpu.VMEM((CHUNKS_PER_W, CHUNK), jnp.int32),
            pltpu.VMEM((CHUNK, D_C), jnp.float32),
            pltpu.VMEM_SHARED((N_NODES_C, D_C), jnp.float32),
            pltpu.SemaphoreType.DMA,
        ],
    )


def _lin_in_body(x_ref, w_ref, b_ref, o_ref):
    o_ref[...] = jnp.dot(x_ref[...], w_ref[...],
                         preferred_element_type=jnp.float32) + b_ref[...]


def _layer_body(h_ref, agg_ref, w1_ref, b1_ref, w2_ref, b2_ref, g_ref, be_ref,
                o_ref):
    z = h_ref[...] + agg_ref[0] + agg_ref[1]
    z = jnp.dot(z, w1_ref[...], preferred_element_type=jnp.float32) + b1_ref[...]
    z = jnp.maximum(z, 0.0)
    z = jnp.dot(z, w2_ref[...], preferred_element_type=jnp.float32) + b2_ref[...]
    mean = jnp.mean(z, axis=0, keepdims=True)
    zc = z - mean
    var = jnp.mean(zc * zc, axis=0, keepdims=True)
    z = zc * lax.rsqrt(var + EPS_BN) * g_ref[...] + be_ref[...]
    o_ref[...] = jnp.maximum(z, 0.0)


def _pool_body(h_ref, batch_ref, wm1_ref, bm1_ref, wm2_ref, bm2_ref, o_ref):
    gids = lax.broadcasted_iota(jnp.int32, (N_GRAPHS_C, N_NODES_C), 0)
    mask = (gids == batch_ref[...]).astype(jnp.float32)
    pooled = jnp.dot(mask, h_ref[...], preferred_element_type=jnp.float32)
    p = jnp.maximum(
        jnp.dot(pooled, wm1_ref[...], preferred_element_type=jnp.float32)
        + bm1_ref[...], 0.0)
    o_ref[...] = jnp.dot(p, wm2_ref[...],
                         preferred_element_type=jnp.float32) + bm2_ref[...]


def kernel(x, edge_index, batch, W_in, b_in, W1, b1, W2, b2, gamma, beta,
           Wm1, bm1, Wm2, bm2):
    n_layers = W1.shape[0]
    src = edge_index[0].astype(jnp.int32).reshape(NW, CHUNKS_PER_W, CHUNK)
    dst = edge_index[1].astype(jnp.int32).reshape(NW, CHUNKS_PER_W, CHUNK)
    batch_i = batch.astype(jnp.int32).reshape(1, N_NODES_C)
    zeros = jnp.zeros((N_NODES_C, D_C), jnp.float32)

    h = pl.pallas_call(
        _lin_in_body,
        out_shape=jax.ShapeDtypeStruct((N_NODES_C, D_C), jnp.float32),
    )(x, W_in, b_in.reshape(1, D_C))

    layer_tc = pl.pallas_call(
        _layer_body,
        out_shape=jax.ShapeDtypeStruct((N_NODES_C, D_C), jnp.float32),
    )

    agg_sc = _make_agg_sc()
    for l in range(n_layers):
        agg = agg_sc(h, src, dst, zeros)
        h = layer_tc(h, agg, W1[l], b1[l].reshape(1, D_C), W2[l],
                     b2[l].reshape(1, D_C), gamma[l].reshape(1, D_C),
                     beta[l].reshape(1, D_C))

    out = pl.pallas_call(
        _pool_body,
        out_shape=jax.ShapeDtypeStruct((N_GRAPHS_C, 1), jnp.float32),
    )(h, batch_i, Wm1, bm1.reshape(1, D_C), Wm2, bm2.reshape(1, 1))
    return out


# speedup vs baseline: 11.3998x; 1.7639x over previous
import functools

import jax
import jax.numpy as jnp
from jax import lax
from jax.experimental import pallas as pl
from jax.experimental.pallas import tpu as pltpu
from jax.experimental.pallas import tpu_sc as plsc

N_NODES_C = 10000
N_EDGES_C = 320000
D_C = 128
N_GRAPHS_C = 64
EPS_BN = 1e-5

NC = 2
NS = 16
NW = NC * NS
EDGES_PER_W = N_EDGES_C // NW
CHUNK = 80
CHUNKS_PER_W = EDGES_PER_W // CHUNK
NGROUPS = 5
GCHUNKS = CHUNKS_PER_W // NGROUPS
STRIPE_OFF = 624
STRIPE_LEN = 640
NBUF = 3


def _agg_body(h_hbm, src_hbm, dst_hbm, zeros_hbm, out_hbm,
              src_v, dst_v, rows_v, acc_sh, sem):
    c = lax.axis_index("c")
    s = lax.axis_index("s")
    wid = s * NC + c

    pltpu.sync_copy(zeros_hbm.at[pl.ds(s * STRIPE_OFF, STRIPE_LEN)],
                    acc_sh.at[pl.ds(s * STRIPE_OFF, STRIPE_LEN)])

    plsc.subcore_barrier()

    def group(g, carry):
        pltpu.sync_copy(src_hbm.at[wid, g], src_v)
        pltpu.sync_copy(dst_hbm.at[wid, g], dst_v)

        for k in range(NBUF):
            pltpu.async_copy(h_hbm.at[src_v.at[k]], rows_v.at[k], sem.at[k])

        def body(j, carry):
            b = lax.rem(j, NBUF)
            pltpu.make_async_copy(h_hbm.at[src_v.at[j]], rows_v.at[b],
                                  sem.at[b]).wait()
            pltpu.sync_copy(rows_v.at[b], acc_sh.at[dst_v.at[j]], add=True)

            @pl.when(j + NBUF < GCHUNKS)
            def _():
                pltpu.async_copy(h_hbm.at[src_v.at[j + NBUF]], rows_v.at[b],
                                 sem.at[b])

            return carry

        return lax.fori_loop(0, GCHUNKS, body, carry, unroll=False)

    lax.fori_loop(0, NGROUPS, group, 0, unroll=False)

    plsc.subcore_barrier()

    pltpu.sync_copy(acc_sh.at[pl.ds(s * STRIPE_OFF, STRIPE_LEN)],
                    out_hbm.at[c, pl.ds(s * STRIPE_OFF, STRIPE_LEN)])


@functools.lru_cache(maxsize=None)
def _make_agg_sc():
    return pl.kernel(
        _agg_body,
        out_type=jax.ShapeDtypeStruct((NC, N_NODES_C, D_C), jnp.float32),
        mesh=plsc.VectorSubcoreMesh(core_axis_name="c", subcore_axis_name="s"),
        scratch_types=[
            pltpu.VMEM((GCHUNKS, CHUNK), jnp.int32),
            pltpu.VMEM((GCHUNKS, CHUNK), jnp.int32),
            pltpu.VMEM((NBUF, CHUNK, D_C), jnp.float32),
            pltpu.VMEM_SHARED((N_NODES_C, D_C), jnp.float32),
            pltpu.SemaphoreType.DMA((NBUF,)),
        ],
    )


def _lin_in_body(x_ref, w_ref, b_ref, o_ref):
    o_ref[...] = jnp.dot(x_ref[...], w_ref[...],
                         preferred_element_type=jnp.float32) + b_ref[...]


def _layer_body(h_ref, agg_ref, w1_ref, b1_ref, w2_ref, b2_ref, g_ref, be_ref,
                o_ref):
    z = h_ref[...] + agg_ref[0] + agg_ref[1]
    z = jnp.dot(z, w1_ref[...], preferred_element_type=jnp.float32) + b1_ref[...]
    z = jnp.maximum(z, 0.0)
    z = jnp.dot(z, w2_ref[...], preferred_element_type=jnp.float32) + b2_ref[...]
    mean = jnp.mean(z, axis=0, keepdims=True)
    zc = z - mean
    var = jnp.mean(zc * zc, axis=0, keepdims=True)
    z = zc * lax.rsqrt(var + EPS_BN) * g_ref[...] + be_ref[...]
    o_ref[...] = jnp.maximum(z, 0.0)


def _pool_body(h_ref, batch_ref, wm1_ref, bm1_ref, wm2_ref, bm2_ref, o_ref):
    gids = lax.broadcasted_iota(jnp.int32, (N_GRAPHS_C, N_NODES_C), 0)
    mask = (gids == batch_ref[...]).astype(jnp.float32)
    pooled = jnp.dot(mask, h_ref[...], preferred_element_type=jnp.float32)
    p = jnp.maximum(
        jnp.dot(pooled, wm1_ref[...], preferred_element_type=jnp.float32)
        + bm1_ref[...], 0.0)
    o_ref[...] = jnp.dot(p, wm2_ref[...],
                         preferred_element_type=jnp.float32) + bm2_ref[...]


def kernel(x, edge_index, batch, W_in, b_in, W1, b1, W2, b2, gamma, beta,
           Wm1, bm1, Wm2, bm2):
    n_layers = W1.shape[0]
    src = edge_index[0].astype(jnp.int32).reshape(NW, NGROUPS, GCHUNKS, CHUNK)
    dst = edge_index[1].astype(jnp.int32).reshape(NW, NGROUPS, GCHUNKS, CHUNK)
    batch_i = batch.astype(jnp.int32).reshape(1, N_NODES_C)
    zeros = jnp.zeros((N_NODES_C, D_C), jnp.float32)

    h = pl.pallas_call(
        _lin_in_body,
        out_shape=jax.ShapeDtypeStruct((N_NODES_C, D_C), jnp.float32),
    )(x, W_in, b_in.reshape(1, D_C))

    layer_tc = pl.pallas_call(
        _layer_body,
        out_shape=jax.ShapeDtypeStruct((N_NODES_C, D_C), jnp.float32),
    )

    agg_sc = _make_agg_sc()
    for l in range(n_layers):
        agg = agg_sc(h, src, dst, zeros)
        h = layer_tc(h, agg, W1[l], b1[l].reshape(1, D_C), W2[l],
                     b2[l].reshape(1, D_C), gamma[l].reshape(1, D_C),
                     beta[l].reshape(1, D_C))

    out = pl.pallas_call(
        _pool_body,
        out_shape=jax.ShapeDtypeStruct((N_GRAPHS_C, 1), jnp.float32),
    )(h, batch_i, Wm1, bm1.reshape(1, D_C), Wm2, bm2.reshape(1, 1))
    return out
